# Initial kernel scaffold; baseline (speedup 1.0000x reference)
import jax
import jax.numpy as jnp
from jax.experimental import pallas as pl


def kernel(x, codebooks, post_scale, post_bias, conv_w, conv_b):
    raise NotImplementedError("write your pallas kernel here")


# speedup vs baseline: 2.3339x; 2.3339x over previous
import jax
import jax.numpy as jnp
from jax import lax
from jax.experimental import pallas as pl
from jax.experimental.pallas import tpu as pltpu
from jax.experimental.pallas import tpu_sc as plsc

_NQ = 8
_K = 1024
_D = 8
_N = 16 * 2048
_BLK = 4096
_GRID = _N // _BLK
_COMMIT_SCALE = 0.25 / (_N * _D)

_NW = 32
_VPW = _N // _NW
_L = 16
_IC = 128
_NCH = _VPW // _IC


def _tc_stage_body(r16_ref, cb_ref, codes_ref):
    rblk = r16_ref[...]
    rT = jnp.transpose(rblk[:, :_D])
    cb_q = cb_ref[...]
    cbsq = jnp.sum(cb_q * cb_q, axis=1, keepdims=True)
    ch = cbsq.astype(jnp.bfloat16).astype(jnp.float32)
    c1 = cbsq - ch
    cm = c1.astype(jnp.bfloat16).astype(jnp.float32)
    cl = c1 - cm
    A = jnp.concatenate([cb_q * jnp.float32(-2.0), ch, cm, cl], axis=1)
    Bm = jnp.concatenate([rT, jnp.ones((3, _BLK), jnp.float32)], axis=0)
    scores = jnp.dot(A, Bm, preferred_element_type=jnp.float32)
    idx = jnp.argmin(scores, axis=0).reshape(1, 1, _BLK)
    codes_ref[...] = idx


def _tc_stage(r16, cb_q):
    return pl.pallas_call(
        _tc_stage_body,
        grid=(_GRID,),
        in_specs=[
            pl.BlockSpec((_BLK, _L), lambda i: (i, 0)),
            pl.BlockSpec((_K, _D), lambda i: (0, 0)),
        ],
        out_specs=pl.BlockSpec((1, 1, _BLK), lambda i: (i, 0, 0)),
        out_shape=jax.ShapeDtypeStruct((_GRID, 1, _BLK), jnp.int32),
    )(r16, cb_q)


def _sc_stage_body(r_hbm, tbl_hbm, idx_hbm, qsum_hbm, s_hbm, b_hbm,
                   rout_hbm, qout_hbm, loss_hbm,
                   idx_v, r_v, q_v, e_v, s_v, b_v, acc_v, sem):
    c = lax.axis_index("c")
    s = lax.axis_index("s")
    wid = s * 2 + c
    base = wid * _VPW
    pltpu.sync_copy(idx_hbm.at[pl.ds(wid * _NCH, _NCH)], idx_v)
    pltpu.sync_copy(r_hbm.at[pl.ds(base, _VPW)], r_v)
    pltpu.sync_copy(qsum_hbm.at[pl.ds(base, _VPW)], q_v)
    pltpu.sync_copy(s_hbm, s_v)
    pltpu.sync_copy(b_hbm, b_v)
    for ci in range(_NCH):
        pltpu.async_copy(tbl_hbm.at[idx_v.at[ci]],
                         e_v.at[pl.ds(ci * _IC, _IC)], sem).wait()
    sv = s_v[...]
    bv = b_v[...]

    def body(g, acc):
        e = e_v[g]
        r = r_v[g]
        rn = r - e
        r_v[g] = rn
        q0 = q_v[g]
        q_v[g] = q0 + (e * sv + bv)
        t = rn - e
        return acc + t * t

    acc = lax.fori_loop(0, _VPW, body, jnp.zeros((_L,), jnp.float32))
    acc_v[...] = acc
    pltpu.sync_copy(r_v, rout_hbm.at[pl.ds(base, _VPW)])
    pltpu.sync_copy(q_v, qout_hbm.at[pl.ds(base, _VPW)])
    pltpu.sync_copy(acc_v, loss_hbm.at[wid])


_sc_stage = pl.kernel(
    _sc_stage_body,
    out_type=[
        jax.ShapeDtypeStruct((_N, _L), jnp.float32),
        jax.ShapeDtypeStruct((_N, _L), jnp.float32),
        jax.ShapeDtypeStruct((_NW, _L), jnp.float32),
    ],
    mesh=plsc.VectorSubcoreMesh(core_axis_name="c", subcore_axis_name="s"),
    compiler_params=pltpu.CompilerParams(use_tc_tiling_on_sc=False),
    scratch_types=[
        pltpu.VMEM((_NCH, _IC), jnp.int32),
        pltpu.VMEM((_VPW, _L), jnp.float32),
        pltpu.VMEM((_VPW, _L), jnp.float32),
        pltpu.VMEM((_VPW, _L), jnp.float32),
        pltpu.VMEM((_L,), jnp.float32),
        pltpu.VMEM((_L,), jnp.float32),
        pltpu.VMEM((_L,), jnp.float32),
        pltpu.SemaphoreType.DMA,
    ],
)


def _tc_final_body(qsum_ref, cw_ref, cbias_ref, lp_ref, outT_ref, loss_ref):
    b = pl.program_id(0)
    qsT = jnp.transpose(qsum_ref[...][:, :_D])
    outT = jnp.dot(cw_ref[...], qsT,
                   preferred_element_type=jnp.float32,
                   precision=jax.lax.Precision.HIGHEST) + cbias_ref[...]
    outT_ref[...] = outT

    @pl.when(b == 0)
    def _():
        loss_ref[...] = jnp.sum(lp_ref[...]).reshape(1, 1) * _COMMIT_SCALE


def _tc_final(qsum16, conv_w, cbias, lossparts):
    return pl.pallas_call(
        _tc_final_body,
        grid=(_GRID,),
        in_specs=[
            pl.BlockSpec((_BLK, _L), lambda i: (i, 0)),
            pl.BlockSpec((_D, _D), lambda i: (0, 0)),
            pl.BlockSpec((_D, 1), lambda i: (0, 0)),
            pl.BlockSpec((_NQ * _NW, _L), lambda i: (0, 0)),
        ],
        out_specs=[
            pl.BlockSpec((_D, _BLK), lambda i: (0, i)),
            pl.BlockSpec((1, 1), lambda i: (0, 0)),
        ],
        out_shape=[
            jax.ShapeDtypeStruct((_D, _N), jnp.float32),
            jax.ShapeDtypeStruct((1, 1), jnp.float32),
        ],
    )(qsum16, conv_w, cbias, lossparts)


def kernel(x, codebooks, post_scale, post_bias, conv_w, conv_b):
    B, T, D = x.shape
    r16 = jnp.concatenate(
        [x.reshape(-1, D), jnp.zeros((_N, _L - _D), jnp.float32)], axis=1)
    tbl16 = jnp.concatenate(
        [codebooks, jnp.zeros((_NQ, _K, _L - _D), jnp.float32)], axis=2)
    cbias = conv_b.reshape(D, 1)
    qsum16 = jnp.zeros((_N, _L), jnp.float32)
    codes_rows = []
    lossparts = []
    for q in range(_NQ):
        idx3 = _tc_stage(r16, codebooks[q])
        idxq = idx3.reshape(_N)
        sfull = jnp.full((_L,), post_scale[q], jnp.float32)
        bfull = jnp.full((_L,), post_bias[q], jnp.float32)
        r16, qsum16, lp = _sc_stage(r16, tbl16[q], idxq.reshape(_NW * _NCH, _IC),
                                    qsum16, sfull, bfull)
        codes_rows.append(idxq)
        lossparts.append(lp)
    outT, loss = _tc_final(qsum16, conv_w, cbias,
                           jnp.concatenate(lossparts, axis=0))
    quantized = outT.T.reshape(B, T, D)
    codes = jnp.stack(codes_rows, axis=0).reshape(_NQ, B, T)
    return quantized, loss[0, 0], codes


# speedup vs baseline: 2.4373x; 1.0443x over previous
import jax
import jax.numpy as jnp
from jax import lax
from jax.experimental import pallas as pl
from jax.experimental.pallas import tpu as pltpu
from jax.experimental.pallas import tpu_sc as plsc

_NQ = 8
_K = 1024
_D = 8
_N = 16 * 2048
_BLK = 4096
_GRID = _N // _BLK
_COMMIT_SCALE = 0.25 / (_N * _D)

_NW = 32
_VPW = _N // _NW
_L = 16
_IC = 128
_NCH = _VPW // _IC


def _tc_stage_body(r16_ref, cb_ref, codes_ref):
    rblk = r16_ref[...]
    rT = jnp.transpose(rblk[:, :_D])
    cb_q = cb_ref[...]
    cbsq = jnp.sum(cb_q * cb_q, axis=1, keepdims=True)
    ch = cbsq.astype(jnp.bfloat16).astype(jnp.float32)
    c1 = cbsq - ch
    cm = c1.astype(jnp.bfloat16).astype(jnp.float32)
    cl = c1 - cm
    A = jnp.concatenate([cb_q * jnp.float32(-2.0), ch, cm, cl], axis=1)
    Bm = jnp.concatenate([rT, jnp.ones((3, _BLK), jnp.float32)], axis=0)
    scores = jnp.dot(A, Bm, preferred_element_type=jnp.float32)
    idx = jnp.argmin(scores, axis=0).reshape(1, 1, _BLK)
    codes_ref[...] = idx


def _tc_stage(r16, cb_q):
    return pl.pallas_call(
        _tc_stage_body,
        grid=(_GRID,),
        in_specs=[
            pl.BlockSpec((_BLK, _L), lambda i: (i, 0)),
            pl.BlockSpec((_K, _D), lambda i: (0, 0)),
        ],
        out_specs=pl.BlockSpec((1, 1, _BLK), lambda i: (i, 0, 0)),
        out_shape=jax.ShapeDtypeStruct((_GRID, 1, _BLK), jnp.int32),
    )(r16, cb_q)


def _sc_stage_body(r_hbm, tbl_hbm, idx_hbm, qsum_hbm, s_hbm, b_hbm,
                   rout_hbm, qout_hbm, loss_hbm,
                   idx_v, r_v, q_v, e_v, s_v, b_v, acc_v, sem):
    c = lax.axis_index("c")
    s = lax.axis_index("s")
    wid = s * 2 + c
    base = wid * _VPW
    pltpu.sync_copy(idx_hbm.at[pl.ds(wid * _NCH, _NCH)], idx_v)
    pltpu.sync_copy(r_hbm.at[pl.ds(base, _VPW)], r_v)
    pltpu.sync_copy(qsum_hbm.at[pl.ds(base, _VPW)], q_v)
    pltpu.sync_copy(s_hbm, s_v)
    pltpu.sync_copy(b_hbm, b_v)
    handles = [
        pltpu.async_copy(tbl_hbm.at[idx_v.at[ci]],
                         e_v.at[pl.ds(ci * _IC, _IC)], sem)
        for ci in range(_NCH)
    ]
    for h in handles:
        h.wait()
    sv = s_v[...]
    bv = b_v[...]

    _U = 8

    def body(g, accs):
        a0, a1, a2, a3 = accs
        acc_list = [a0, a1, a2, a3]
        start = g * _U
        for u in range(_U):
            row = start + u
            e = e_v[row]
            r = r_v[row]
            rn = r - e
            r_v[row] = rn
            q0 = q_v[row]
            q_v[row] = q0 + (e * sv + bv)
            t = rn - e
            acc_list[u % 4] = acc_list[u % 4] + t * t
        return tuple(acc_list)

    z = jnp.zeros((_L,), jnp.float32)
    a0, a1, a2, a3 = lax.fori_loop(0, _VPW // _U, body, (z, z, z, z))
    acc_v[...] = (a0 + a1) + (a2 + a3)
    pltpu.sync_copy(r_v, rout_hbm.at[pl.ds(base, _VPW)])
    pltpu.sync_copy(q_v, qout_hbm.at[pl.ds(base, _VPW)])
    pltpu.sync_copy(acc_v, loss_hbm.at[wid])


_sc_stage = pl.kernel(
    _sc_stage_body,
    out_type=[
        jax.ShapeDtypeStruct((_N, _L), jnp.float32),
        jax.ShapeDtypeStruct((_N, _L), jnp.float32),
        jax.ShapeDtypeStruct((_NW, _L), jnp.float32),
    ],
    mesh=plsc.VectorSubcoreMesh(core_axis_name="c", subcore_axis_name="s"),
    compiler_params=pltpu.CompilerParams(use_tc_tiling_on_sc=False),
    scratch_types=[
        pltpu.VMEM((_NCH, _IC), jnp.int32),
        pltpu.VMEM((_VPW, _L), jnp.float32),
        pltpu.VMEM((_VPW, _L), jnp.float32),
        pltpu.VMEM((_VPW, _L), jnp.float32),
        pltpu.VMEM((_L,), jnp.float32),
        pltpu.VMEM((_L,), jnp.float32),
        pltpu.VMEM((_L,), jnp.float32),
        pltpu.SemaphoreType.DMA,
    ],
)


def _tc_final_body(qsum_ref, cw_ref, cbias_ref, lp_ref, outT_ref, loss_ref):
    b = pl.program_id(0)
    qsT = jnp.transpose(qsum_ref[...][:, :_D])
    outT = jnp.dot(cw_ref[...], qsT,
                   preferred_element_type=jnp.float32,
                   precision=jax.lax.Precision.HIGHEST) + cbias_ref[...]
    outT_ref[...] = outT

    @pl.when(b == 0)
    def _():
        loss_ref[...] = jnp.sum(lp_ref[...]).reshape(1, 1) * _COMMIT_SCALE


def _tc_final(qsum16, conv_w, cbias, lossparts):
    return pl.pallas_call(
        _tc_final_body,
        grid=(_GRID,),
        in_specs=[
            pl.BlockSpec((_BLK, _L), lambda i: (i, 0)),
            pl.BlockSpec((_D, _D), lambda i: (0, 0)),
            pl.BlockSpec((_D, 1), lambda i: (0, 0)),
            pl.BlockSpec((_NQ * _NW, _L), lambda i: (0, 0)),
        ],
        out_specs=[
            pl.BlockSpec((_D, _BLK), lambda i: (0, i)),
            pl.BlockSpec((1, 1), lambda i: (0, 0)),
        ],
        out_shape=[
            jax.ShapeDtypeStruct((_D, _N), jnp.float32),
            jax.ShapeDtypeStruct((1, 1), jnp.float32),
        ],
    )(qsum16, conv_w, cbias, lossparts)


def kernel(x, codebooks, post_scale, post_bias, conv_w, conv_b):
    B, T, D = x.shape
    r16 = jnp.concatenate(
        [x.reshape(-1, D), jnp.zeros((_N, _L - _D), jnp.float32)], axis=1)
    tbl16 = jnp.concatenate(
        [codebooks, jnp.zeros((_NQ, _K, _L - _D), jnp.float32)], axis=2)
    cbias = conv_b.reshape(D, 1)
    qsum16 = jnp.zeros((_N, _L), jnp.float32)
    codes_rows = []
    lossparts = []
    for q in range(_NQ):
        idx3 = _tc_stage(r16, codebooks[q])
        idxq = idx3.reshape(_N)
        sfull = jnp.full((_L,), post_scale[q], jnp.float32)
        bfull = jnp.full((_L,), post_bias[q], jnp.float32)
        r16, qsum16, lp = _sc_stage(r16, tbl16[q], idxq.reshape(_NW * _NCH, _IC),
                                    qsum16, sfull, bfull)
        codes_rows.append(idxq)
        lossparts.append(lp)
    outT, loss = _tc_final(qsum16, conv_w, cbias,
                           jnp.concatenate(lossparts, axis=0))
    quantized = outT.T.reshape(B, T, D)
    codes = jnp.stack(codes_rows, axis=0).reshape(_NQ, B, T)
    return quantized, loss[0, 0], codes


# speedup vs baseline: 5.6095x; 2.3015x over previous
import jax
import jax.numpy as jnp
from jax import lax
from jax.experimental import pallas as pl
from jax.experimental.pallas import tpu as pltpu

_NQ = 8
_K = 1024
_D = 8
_N = 16 * 2048
_BLK = 4096
_GRID = _N // _BLK
_COMMIT_SCALE = 0.25 / (_N * _D)


def _rvq_body(xT_ref, cb_ref, cbT_ref, ps_ref, pb_ref, cw_ref, cbias_ref,
              outT_ref, codes_ref, loss_ref):
    b = pl.program_id(0)
    rT = xT_ref[...]
    qsumT = jnp.zeros((_D, _BLK), jnp.float32)
    loss = jnp.float32(0.0)
    iota = lax.broadcasted_iota(jnp.int32, (_K, _BLK), 0)
    for q in range(_NQ):
        cb_q = cb_ref[q]
        cbT_q = cbT_ref[q]
        cbsq = jnp.sum(cb_q * cb_q, axis=1, keepdims=True)
        ch = cbsq.astype(jnp.bfloat16).astype(jnp.float32)
        c1 = cbsq - ch
        cm = c1.astype(jnp.bfloat16).astype(jnp.float32)
        cl = c1 - cm
        A = jnp.concatenate([cb_q * jnp.float32(-2.0), ch, cm, cl], axis=1)
        Bm = jnp.concatenate([rT, jnp.ones((3, _BLK), jnp.float32)], axis=0)
        scores = jnp.dot(A, Bm, preferred_element_type=jnp.float32)
        idx = jnp.argmin(scores, axis=0).reshape(1, _BLK)
        codes_ref[q:q + 1, :] = idx
        onehot = (iota == idx).astype(jnp.bfloat16)
        chi = cbT_q.astype(jnp.bfloat16)
        r1 = cbT_q - chi.astype(jnp.float32)
        cmid = r1.astype(jnp.bfloat16)
        clo = (r1 - cmid.astype(jnp.float32)).astype(jnp.bfloat16)
        cstack = jnp.concatenate([chi, cmid, clo], axis=0)
        emb3 = jnp.dot(cstack, onehot,
                       preferred_element_type=jnp.float32)
        embT = (emb3[0:_D] + emb3[_D:2 * _D]) + emb3[2 * _D:3 * _D]
        s = ps_ref[q:q + 1, :]
        t = pb_ref[q:q + 1, :]
        qsumT = qsumT + (embT * s + t)
        rT = rT - embT
        diff = rT - embT
        loss = loss + jnp.sum(diff * diff)
    outT = jnp.dot(cw_ref[...], qsumT,
                   preferred_element_type=jnp.float32,
                   precision=jax.lax.Precision.HIGHEST) + cbias_ref[...]
    outT_ref[...] = outT
    prev = jnp.where(b == 0, jnp.zeros((1, 1), jnp.float32), loss_ref[...])
    loss_ref[...] = prev + loss * _COMMIT_SCALE


def kernel(x, codebooks, post_scale, post_bias, conv_w, conv_b):
    B, T, D = x.shape
    xT = x.reshape(-1, D).T
    cbT = codebooks.transpose(0, 2, 1)
    ps = post_scale.reshape(_NQ, 1)
    pb = post_bias.reshape(_NQ, 1)
    cbias = conv_b.reshape(D, 1)
    outT, codes, loss = pl.pallas_call(
        _rvq_body,
        grid=(_GRID,),
        in_specs=[
            pl.BlockSpec((_D, _BLK), lambda i: (0, i)),
            pl.BlockSpec((_NQ, _K, _D), lambda i: (0, 0, 0)),
            pl.BlockSpec((_NQ, _D, _K), lambda i: (0, 0, 0)),
            pl.BlockSpec((_NQ, 1), lambda i: (0, 0)),
            pl.BlockSpec((_NQ, 1), lambda i: (0, 0)),
            pl.BlockSpec((_D, _D), lambda i: (0, 0)),
            pl.BlockSpec((_D, 1), lambda i: (0, 0)),
        ],
        out_specs=[
            pl.BlockSpec((_D, _BLK), lambda i: (0, i)),
            pl.BlockSpec((_NQ, _BLK), lambda i: (0, i)),
            pl.BlockSpec((1, 1), lambda i: (0, 0)),
        ],
        out_shape=[
            jax.ShapeDtypeStruct((_D, _N), jnp.float32),
            jax.ShapeDtypeStruct((_NQ, _N), jnp.int32),
            jax.ShapeDtypeStruct((1, 1), jnp.float32),
        ],
        interpret=False,
    )(xT, codebooks, cbT, ps, pb, conv_w, cbias)
    quantized = outT.T.reshape(B, T, D)
    return quantized, loss[0, 0], codes.reshape(_NQ, B, T)
